# Initial kernel scaffold; baseline (speedup 1.0000x reference)
import jax
import jax.numpy as jnp
from jax.experimental import pallas as pl


def kernel(x, edge_index, edge_attr, pos, W_message, b_message):
    raise NotImplementedError("write your pallas kernel here")


# speedup vs baseline: 10.6030x; 10.6030x over previous
import functools
import math

import jax
import jax.numpy as jnp
from jax import lax
from jax.experimental import pallas as pl
from jax.experimental.pallas import tpu as pltpu
from jax.experimental.pallas import tpu_sc as plsc

N = 10000
E = 160000
IN_C = 128
OUT_C = 128
NA = 7
FD = 2 * IN_C + 1

NS = 16
CHUNK = 80
EPT = E // NS
NCHUNK = EPT // CHUNK
CNT_PAD = 71680
DEG_PAD = 10240
ROWS_N = DEG_PAD // NS

_T2 = [float(math.cos(k * math.pi / NA) ** 2) for k in range(1, NA)]


def _sc_edge_kernel(row_ref, col_ref, ea_ref, px_ref, py_ref, pz_ref, z_ref, u_ref,
                    z2d_ref, z1d_ref,
                    g2_out, sxu_out, cnt_out, sab_out, deg_out, sea_out,
                    acc, s1, s2,
                    row_v, col_v, zidx_v, ridx_v, ea_v, ones_v, rows_v,
                    pxr_v, pyr_v, pzr_v, pxc_v, pyc_v, pzc_v, sem):
    cid = lax.axis_index("c")
    sid = lax.axis_index("s")

    pltpu.sync_copy(z2d_ref, acc.at[pl.ds(sid * ROWS_N, ROWS_N), :])
    pltpu.sync_copy(z1d_ref.at[pl.ds(0, CNT_PAD // NS)],
                    s1.at[pl.ds(sid * (CNT_PAD // NS), CNT_PAD // NS)])
    pltpu.sync_copy(z1d_ref.at[pl.ds(0, CNT_PAD // NS)],
                    s2.at[pl.ds(sid * (CNT_PAD // NS), CNT_PAD // NS)])

    for g in range(CHUNK // 16):
        ones_v[pl.ds(g * 16, 16)] = jnp.ones((16,), jnp.float32)

    plsc.subcore_barrier()

    base0 = sid * EPT

    def chunk_body(i, carry):
        base = base0 + i * CHUNK
        pltpu.sync_copy(row_ref.at[pl.ds(base, CHUNK)], row_v)
        pltpu.sync_copy(col_ref.at[pl.ds(base, CHUNK)], col_v)
        pltpu.sync_copy(ea_ref.at[pl.ds(base, CHUNK)], ea_v)

        @pl.when(cid == 0)
        def _core0():
            c1 = pltpu.async_copy(px_ref.at[row_v], pxr_v, sem)
            c2 = pltpu.async_copy(py_ref.at[row_v], pyr_v, sem)
            c3 = pltpu.async_copy(pz_ref.at[row_v], pzr_v, sem)
            c4 = pltpu.async_copy(px_ref.at[col_v], pxc_v, sem)
            c5 = pltpu.async_copy(py_ref.at[col_v], pyc_v, sem)
            c6 = pltpu.async_copy(pz_ref.at[col_v], pzc_v, sem)
            c1.wait(); c2.wait(); c3.wait(); c4.wait(); c5.wait(); c6.wait()
            for g in range(CHUNK // 16):
                r16 = row_v[pl.ds(g * 16, 16)]
                c16 = col_v[pl.ds(g * 16, 16)]
                dx = pxc_v[pl.ds(g * 16, 16)] - pxr_v[pl.ds(g * 16, 16)]
                dy = pyc_v[pl.ds(g * 16, 16)] - pyr_v[pl.ds(g * 16, 16)]
                dz = pzc_v[pl.ds(g * 16, 16)] - pzr_v[pl.ds(g * 16, 16)]
                s = dx * dx + dy * dy + dz * dz
                vx2 = dx * dx
                neg = dx < 0.0
                bins = jnp.zeros((16,), jnp.int32)
                for k in range(NA - 1):
                    if k < 3:
                        hit = neg | (vx2 < _T2[k] * s)
                    else:
                        hit = neg & (vx2 > _T2[k] * s)
                    bins = bins + hit.astype(jnp.int32)
                bins = jnp.where(s == 0.0, 3, bins)
                zidx_v[pl.ds(g * 16, 16)] = c16 * NA + bins
                ridx_v[pl.ds(g * 16, 16)] = r16 * NA + bins
            pltpu.async_copy(z_ref.at[zidx_v], rows_v, sem).wait()
            pltpu.sync_copy(rows_v, acc.at[row_v], add=True)
            pltpu.sync_copy(ones_v, s1.at[ridx_v], add=True)
            pltpu.sync_copy(ea_v, s2.at[ridx_v], add=True)

        @pl.when(cid == 1)
        def _core1():
            pltpu.async_copy(u_ref.at[row_v], rows_v, sem).wait()
            pltpu.sync_copy(rows_v, acc.at[col_v], add=True)
            pltpu.sync_copy(ones_v, s1.at[col_v], add=True)
            pltpu.sync_copy(ea_v, s2.at[col_v], add=True)

        return carry

    lax.fori_loop(0, NCHUNK, chunk_body, 0)

    plsc.subcore_barrier()

    @pl.when(cid == 0)
    def _out0():
        pltpu.sync_copy(acc.at[pl.ds(sid * ROWS_N, ROWS_N), :],
                        g2_out.at[pl.ds(sid * ROWS_N, ROWS_N), :])
        pltpu.sync_copy(s1.at[pl.ds(sid * (CNT_PAD // NS), CNT_PAD // NS)],
                        cnt_out.at[pl.ds(sid * (CNT_PAD // NS), CNT_PAD // NS)])
        pltpu.sync_copy(s2.at[pl.ds(sid * (CNT_PAD // NS), CNT_PAD // NS)],
                        sab_out.at[pl.ds(sid * (CNT_PAD // NS), CNT_PAD // NS)])

    @pl.when(cid == 1)
    def _out1():
        pltpu.sync_copy(acc.at[pl.ds(sid * ROWS_N, ROWS_N), :],
                        sxu_out.at[pl.ds(sid * ROWS_N, ROWS_N), :])
        pltpu.sync_copy(s1.at[pl.ds(sid * (DEG_PAD // NS), DEG_PAD // NS)],
                        deg_out.at[pl.ds(sid * (DEG_PAD // NS), DEG_PAD // NS)])
        pltpu.sync_copy(s2.at[pl.ds(sid * (DEG_PAD // NS), DEG_PAD // NS)],
                        sea_out.at[pl.ds(sid * (DEG_PAD // NS), DEG_PAD // NS)])


@functools.lru_cache(maxsize=1)
def _sc_edge_built():
    return functools.partial(
        pl.kernel,
        out_type=[
            jax.ShapeDtypeStruct((DEG_PAD, OUT_C), jnp.float32),
            jax.ShapeDtypeStruct((DEG_PAD, OUT_C), jnp.float32),
            jax.ShapeDtypeStruct((CNT_PAD,), jnp.float32),
            jax.ShapeDtypeStruct((CNT_PAD,), jnp.float32),
            jax.ShapeDtypeStruct((DEG_PAD,), jnp.float32),
            jax.ShapeDtypeStruct((DEG_PAD,), jnp.float32),
        ],
        mesh=plsc.VectorSubcoreMesh(core_axis_name="c", subcore_axis_name="s",
                                    num_cores=2, num_subcores=NS),
        compiler_params=pltpu.CompilerParams(needs_layout_passes=False),
        scratch_types=[
            pltpu.VMEM_SHARED((DEG_PAD, OUT_C), jnp.float32),
            pltpu.VMEM_SHARED((CNT_PAD,), jnp.float32),
            pltpu.VMEM_SHARED((CNT_PAD,), jnp.float32),
            pltpu.VMEM((CHUNK,), jnp.int32),
            pltpu.VMEM((CHUNK,), jnp.int32),
            pltpu.VMEM((CHUNK,), jnp.int32),
            pltpu.VMEM((CHUNK,), jnp.int32),
            pltpu.VMEM((CHUNK,), jnp.float32),
            pltpu.VMEM((CHUNK,), jnp.float32),
            pltpu.VMEM((CHUNK, OUT_C), jnp.float32),
            pltpu.VMEM((CHUNK,), jnp.float32),
            pltpu.VMEM((CHUNK,), jnp.float32),
            pltpu.VMEM((CHUNK,), jnp.float32),
            pltpu.VMEM((CHUNK,), jnp.float32),
            pltpu.VMEM((CHUNK,), jnp.float32),
            pltpu.VMEM((CHUNK,), jnp.float32),
            pltpu.SemaphoreType.DMA,
        ],
    )(_sc_edge_kernel)


def _tc_pre_kernel(x_ref, zw_ref, uw_ref, z_out, u_out):
    xb = x_ref[...]
    z_out[...] = jnp.dot(xb, zw_ref[...], preferred_element_type=jnp.float32)
    u_out[...] = jnp.dot(xb, uw_ref[...], preferred_element_type=jnp.float32)


def _tc_pre(x, zw, uw):
    blk = 1000
    grid = N // blk
    return pl.pallas_call(
        _tc_pre_kernel,
        grid=(grid,),
        in_specs=[
            pl.BlockSpec((blk, IN_C), lambda i: (i, 0)),
            pl.BlockSpec((IN_C, NA * OUT_C), lambda i: (0, 0)),
            pl.BlockSpec((IN_C, OUT_C), lambda i: (0, 0)),
        ],
        out_specs=[
            pl.BlockSpec((blk, NA * OUT_C), lambda i: (i, 0)),
            pl.BlockSpec((blk, OUT_C), lambda i: (i, 0)),
        ],
        out_shape=[
            jax.ShapeDtypeStruct((N, NA * OUT_C), jnp.float32),
            jax.ShapeDtypeStruct((N, OUT_C), jnp.float32),
        ],
    )(x, zw, uw)


def _tc_combine_kernel(x_ref, g2_ref, sxu_ref, cnt_ref, sab_ref, deg_ref,
                       sea_ref, w1_ref, ws2_ref, w3_ref, ws3_ref, b_ref,
                       out_ref):
    xb = x_ref[...]
    cnt = cnt_ref[...]
    sab = sab_ref[...]
    deg = deg_ref[...]
    sea = sea_ref[...]
    acc = jnp.dot(xb, ws2_ref[...], preferred_element_type=jnp.float32)
    acc = acc + g2_ref[...] + b_ref[...]
    for b in range(NA):
        yb = jnp.dot(xb, w1_ref[..., b * OUT_C:(b + 1) * OUT_C],
                     preferred_element_type=jnp.float32)
        acc = acc + cnt[:, b:b + 1] * yb
        acc = acc + sab[:, b:b + 1] * w3_ref[b:b + 1, :]
    out_ref[...] = sxu_ref[...] + sea * ws3_ref[...] + deg * acc


def _tc_combine(x, g2, sxu, cnt, sab, deg, sea, w1, ws2t, w3t, ws3, bvec):
    blk = 1000
    grid = N // blk
    return pl.pallas_call(
        _tc_combine_kernel,
        grid=(grid,),
        in_specs=[
            pl.BlockSpec((blk, IN_C), lambda i: (i, 0)),
            pl.BlockSpec((blk, OUT_C), lambda i: (i, 0)),
            pl.BlockSpec((blk, OUT_C), lambda i: (i, 0)),
            pl.BlockSpec((blk, NA), lambda i: (i, 0)),
            pl.BlockSpec((blk, NA), lambda i: (i, 0)),
            pl.BlockSpec((blk, 1), lambda i: (i, 0)),
            pl.BlockSpec((blk, 1), lambda i: (i, 0)),
            pl.BlockSpec((IN_C, NA * OUT_C), lambda i: (0, 0)),
            pl.BlockSpec((IN_C, OUT_C), lambda i: (0, 0)),
            pl.BlockSpec((NA, OUT_C), lambda i: (0, 0)),
            pl.BlockSpec((1, OUT_C), lambda i: (0, 0)),
            pl.BlockSpec((1, OUT_C), lambda i: (0, 0)),
        ],
        out_specs=pl.BlockSpec((blk, OUT_C), lambda i: (i, 0)),
        out_shape=jax.ShapeDtypeStruct((N, OUT_C), jnp.float32),
    )(x, g2, sxu, cnt, sab, deg, sea, w1, ws2t, w3t, ws3, bvec)


def kernel(x, edge_index, edge_attr, pos, W_message, b_message):
    Wr = W_message.reshape(OUT_C, NA, 2, FD)
    Ws = jnp.sum(Wr[:, :, 0, :], axis=1)
    ws1t = Ws[:, :IN_C].T
    ws2t = Ws[:, IN_C:2 * IN_C].T
    ws3 = Ws[:, 2 * IN_C].reshape(1, OUT_C)
    W2 = Wr[:, :, 1, IN_C:2 * IN_C]
    zw = jnp.transpose(W2, (2, 1, 0)).reshape(IN_C, NA * OUT_C)
    W1 = Wr[:, :, 1, :IN_C]
    w1 = jnp.transpose(W1, (2, 1, 0)).reshape(IN_C, NA * OUT_C)
    w3t = Wr[:, :, 1, 2 * IN_C].T
    bvec = b_message.reshape(1, OUT_C)

    ea_flat = edge_attr.reshape(E)
    px = pos[:, 0]
    py = pos[:, 1]
    pz = pos[:, 2]

    z_tab, u_tab = _tc_pre(x, zw, ws1t)
    z_tab = z_tab.reshape(N * NA, OUT_C)

    z2d = jnp.zeros((ROWS_N, OUT_C), jnp.float32)
    z1d = jnp.zeros((CNT_PAD // NS,), jnp.float32)
    g2, sxu, cnt, sab, deg, sea = _sc_edge_built()(
        edge_index[0], edge_index[1], ea_flat, px, py, pz, z_tab, u_tab,
        z2d, z1d)

    g2 = g2[:N]
    sxu = sxu[:N]
    cnt = cnt[:N * NA].reshape(N, NA)
    sab = sab[:N * NA].reshape(N, NA)
    deg = deg[:N].reshape(N, 1)
    sea = sea[:N].reshape(N, 1)

    return _tc_combine(x, g2, sxu, cnt, sab, deg, sea, w1, ws2t, w3t, ws3, bvec)


# speedup vs baseline: 16.6606x; 1.5713x over previous
import functools
import math

import jax
import jax.numpy as jnp
from jax import lax
from jax.experimental import pallas as pl
from jax.experimental.pallas import tpu as pltpu
from jax.experimental.pallas import tpu_sc as plsc

N = 10000
E = 160000
IN_C = 128
OUT_C = 128
NA = 7
FD = 2 * IN_C + 1

NS = 16
CHUNK = 80
EPT = E // NS
NCHUNK = EPT // CHUNK
CNT_PAD = 71680
DEG_PAD = 10240
ROWS_N = DEG_PAD // NS

_T2 = [float(math.cos(k * math.pi / NA) ** 2) for k in range(1, NA)]


def _sc_edge_kernel(pk_ref, px_ref, py_ref, pz_ref, z_ref, u_ref,
                    z2d_ref, z1d_ref,
                    g2_out, sxu_out, cnt_out, sab_out, deg_out, sea_out,
                    acc, s1, s2,
                    pk0, pk1, ea0, ea1, zidx0, zidx1, ridx0, ridx1,
                    pxr0, pyr0, pzr0, pxc0, pyc0, pzc0,
                    pxr1, pyr1, pzr1, pxc1, pyc1, pzc1,
                    rows0, rows1, ones_v,
                    sem_i0, sem_i1, sem_p0, sem_p1, sem_z0, sem_z1, sem_s):
    cid = lax.axis_index("c")
    sid = lax.axis_index("s")

    pltpu.sync_copy(z2d_ref, acc.at[pl.ds(sid * ROWS_N, ROWS_N), :])
    pltpu.sync_copy(z1d_ref.at[pl.ds(0, CNT_PAD // NS)],
                    s1.at[pl.ds(sid * (CNT_PAD // NS), CNT_PAD // NS)])
    pltpu.sync_copy(z1d_ref.at[pl.ds(0, CNT_PAD // NS)],
                    s2.at[pl.ds(sid * (CNT_PAD // NS), CNT_PAD // NS)])

    for g in range(CHUNK // 16):
        ones_v[pl.ds(g * 16, 16)] = jnp.ones((16,), jnp.float32)

    plsc.subcore_barrier()

    cbase = sid * NCHUNK
    B = ((pk0, ea0, zidx0, ridx0, (pxr0, pyr0, pzr0, pxc0, pyc0, pzc0),
          rows0, sem_p0, sem_z0),
         (pk1, ea1, zidx1, ridx1, (pxr1, pyr1, pzr1, pxc1, pyc1, pzc1),
          rows1, sem_p1, sem_z1))

    def issue_pos(b):
        pk, _, _, _, pos, _, sem_p, _ = B[b]
        pxr, pyr, pzr, pxc, pyc, pzc = pos
        return (pltpu.async_copy(px_ref.at[pk.at[0]], pxr, sem_p),
                pltpu.async_copy(py_ref.at[pk.at[0]], pyr, sem_p),
                pltpu.async_copy(pz_ref.at[pk.at[0]], pzr, sem_p),
                pltpu.async_copy(px_ref.at[pk.at[1]], pxc, sem_p),
                pltpu.async_copy(py_ref.at[pk.at[1]], pyc, sem_p),
                pltpu.async_copy(pz_ref.at[pk.at[1]], pzc, sem_p))

    def unpack_ea(b):
        pk, ea_v, _, _, _, _, _, _ = B[b]
        for g in range(CHUNK // 16):
            bits = pk[2, pl.ds(g * 16, 16)]
            ea_v[pl.ds(g * 16, 16)] = plsc.bitcast(bits, jnp.float32)

    def compute_bins(b):
        pk, _, zidx_v, ridx_v, pos, _, _, _ = B[b]
        pxr, pyr, pzr, pxc, pyc, pzc = pos
        for g in range(CHUNK // 16):
            r16 = pk[0, pl.ds(g * 16, 16)]
            c16 = pk[1, pl.ds(g * 16, 16)]
            dx = pxc[pl.ds(g * 16, 16)] - pxr[pl.ds(g * 16, 16)]
            dy = pyc[pl.ds(g * 16, 16)] - pyr[pl.ds(g * 16, 16)]
            dz = pzc[pl.ds(g * 16, 16)] - pzr[pl.ds(g * 16, 16)]
            s = dx * dx + dy * dy + dz * dz
            vx2 = dx * dx
            neg = dx < 0.0
            bins = jnp.zeros((16,), jnp.int32)
            for k in range(NA - 1):
                if k < 3:
                    hit = neg | (vx2 < _T2[k] * s)
                else:
                    hit = neg & (vx2 > _T2[k] * s)
                bins = bins + hit.astype(jnp.int32)
            bins = jnp.where(s == 0.0, 3, bins)
            zidx_v[pl.ds(g * 16, 16)] = c16 * NA + bins
            ridx_v[pl.ds(g * 16, 16)] = r16 * NA + bins

    def scatter_c0(b):
        pk, ea_v, _, ridx_v, _, rows_v, _, _ = B[b]
        return (pltpu.async_copy(rows_v, acc.at[pk.at[0]], sem_s, add=True),
                pltpu.async_copy(ones_v, s1.at[ridx_v], sem_s, add=True),
                pltpu.async_copy(ea_v, s2.at[ridx_v], sem_s, add=True))

    def scatter_c1(b):
        pk, ea_v, _, _, _, rows_v, _, _ = B[b]
        return (pltpu.async_copy(rows_v, acc.at[pk.at[1]], sem_s, add=True),
                pltpu.async_copy(ones_v, s1.at[pk.at[1]], sem_s, add=True),
                pltpu.async_copy(ea_v, s2.at[pk.at[1]], sem_s, add=True))

    def do_c0(la, lb, tail):
        la.wait()
        pa = issue_pos(0)
        if not tail:
            lb.wait()
            pb = issue_pos(1)
        for c in pa:
            c.wait()
        unpack_ea(0)
        compute_bins(0)
        za = pltpu.async_copy(z_ref.at[zidx0], rows0, sem_z0)
        if not tail:
            for c in pb:
                c.wait()
            unpack_ea(1)
            compute_bins(1)
            zb = pltpu.async_copy(z_ref.at[zidx1], rows1, sem_z1)
        za.wait()
        sa = scatter_c0(0)
        if not tail:
            zb.wait()
            sb = scatter_c0(1)
        for c in sa:
            c.wait()
        if not tail:
            for c in sb:
                c.wait()

    def do_c1(la, lb, tail):
        la.wait()
        ua = pltpu.async_copy(u_ref.at[pk0.at[0]], rows0, sem_z0)
        if not tail:
            lb.wait()
            ub = pltpu.async_copy(u_ref.at[pk1.at[0]], rows1, sem_z1)
        ua.wait()
        unpack_ea(0)
        sa = scatter_c1(0)
        if not tail:
            ub.wait()
            unpack_ea(1)
            sb = scatter_c1(1)
        for c in sa:
            c.wait()
        if not tail:
            for c in sb:
                c.wait()

    def pair_body(i, carry):
        la = pltpu.async_copy(pk_ref.at[cbase + 2 * i], pk0, sem_i0)
        lb = pltpu.async_copy(pk_ref.at[cbase + 2 * i + 1], pk1, sem_i1)

        @pl.when(cid == 0)
        def _c0():
            do_c0(la, lb, False)

        @pl.when(cid == 1)
        def _c1():
            do_c1(la, lb, False)

        return carry

    lax.fori_loop(0, NCHUNK // 2, pair_body, 0)

    if NCHUNK % 2:
        lt = pltpu.async_copy(pk_ref.at[cbase + NCHUNK - 1], pk0, sem_i0)

        @pl.when(cid == 0)
        def _t0():
            do_c0(lt, None, True)

        @pl.when(cid == 1)
        def _t1():
            do_c1(lt, None, True)

    plsc.subcore_barrier()

    @pl.when(cid == 0)
    def _out0():
        pltpu.sync_copy(acc.at[pl.ds(sid * ROWS_N, ROWS_N), :],
                        g2_out.at[pl.ds(sid * ROWS_N, ROWS_N), :])
        pltpu.sync_copy(s1.at[pl.ds(sid * (CNT_PAD // NS), CNT_PAD // NS)],
                        cnt_out.at[pl.ds(sid * (CNT_PAD // NS), CNT_PAD // NS)])
        pltpu.sync_copy(s2.at[pl.ds(sid * (CNT_PAD // NS), CNT_PAD // NS)],
                        sab_out.at[pl.ds(sid * (CNT_PAD // NS), CNT_PAD // NS)])

    @pl.when(cid == 1)
    def _out1():
        pltpu.sync_copy(acc.at[pl.ds(sid * ROWS_N, ROWS_N), :],
                        sxu_out.at[pl.ds(sid * ROWS_N, ROWS_N), :])
        pltpu.sync_copy(s1.at[pl.ds(sid * (DEG_PAD // NS), DEG_PAD // NS)],
                        deg_out.at[pl.ds(sid * (DEG_PAD // NS), DEG_PAD // NS)])
        pltpu.sync_copy(s2.at[pl.ds(sid * (DEG_PAD // NS), DEG_PAD // NS)],
                        sea_out.at[pl.ds(sid * (DEG_PAD // NS), DEG_PAD // NS)])


@functools.lru_cache(maxsize=1)
def _sc_edge_built():
    return functools.partial(
        pl.kernel,
        out_type=[
            jax.ShapeDtypeStruct((DEG_PAD, OUT_C), jnp.float32),
            jax.ShapeDtypeStruct((DEG_PAD, OUT_C), jnp.float32),
            jax.ShapeDtypeStruct((CNT_PAD,), jnp.float32),
            jax.ShapeDtypeStruct((CNT_PAD,), jnp.float32),
            jax.ShapeDtypeStruct((DEG_PAD,), jnp.float32),
            jax.ShapeDtypeStruct((DEG_PAD,), jnp.float32),
        ],
        mesh=plsc.VectorSubcoreMesh(core_axis_name="c", subcore_axis_name="s",
                                    num_cores=2, num_subcores=NS),
        compiler_params=pltpu.CompilerParams(needs_layout_passes=False),
        scratch_types=[
            pltpu.VMEM_SHARED((DEG_PAD, OUT_C), jnp.float32),
            pltpu.VMEM_SHARED((CNT_PAD,), jnp.float32),
            pltpu.VMEM_SHARED((CNT_PAD,), jnp.float32),
            pltpu.VMEM((3, CHUNK), jnp.int32),
            pltpu.VMEM((3, CHUNK), jnp.int32),
            pltpu.VMEM((CHUNK,), jnp.float32),
            pltpu.VMEM((CHUNK,), jnp.float32),
            pltpu.VMEM((CHUNK,), jnp.int32),
            pltpu.VMEM((CHUNK,), jnp.int32),
            pltpu.VMEM((CHUNK,), jnp.int32),
            pltpu.VMEM((CHUNK,), jnp.int32),
            pltpu.VMEM((CHUNK,), jnp.float32),
            pltpu.VMEM((CHUNK,), jnp.float32),
            pltpu.VMEM((CHUNK,), jnp.float32),
            pltpu.VMEM((CHUNK,), jnp.float32),
            pltpu.VMEM((CHUNK,), jnp.float32),
            pltpu.VMEM((CHUNK,), jnp.float32),
            pltpu.VMEM((CHUNK,), jnp.float32),
            pltpu.VMEM((CHUNK,), jnp.float32),
            pltpu.VMEM((CHUNK,), jnp.float32),
            pltpu.VMEM((CHUNK,), jnp.float32),
            pltpu.VMEM((CHUNK,), jnp.float32),
            pltpu.VMEM((CHUNK,), jnp.float32),
            pltpu.VMEM((CHUNK, OUT_C), jnp.float32),
            pltpu.VMEM((CHUNK, OUT_C), jnp.float32),
            pltpu.VMEM((CHUNK,), jnp.float32),
            pltpu.SemaphoreType.DMA,
            pltpu.SemaphoreType.DMA,
            pltpu.SemaphoreType.DMA,
            pltpu.SemaphoreType.DMA,
            pltpu.SemaphoreType.DMA,
            pltpu.SemaphoreType.DMA,
            pltpu.SemaphoreType.DMA,
        ],
    )(_sc_edge_kernel)


def _tc_pre_kernel(x_ref, zw_ref, uw_ref, z_out, u_out):
    xb = x_ref[...]
    z_out[...] = jnp.dot(xb, zw_ref[...], preferred_element_type=jnp.float32)
    u_out[...] = jnp.dot(xb, uw_ref[...], preferred_element_type=jnp.float32)


def _tc_pre(x, zw, uw):
    blk = 1000
    grid = N // blk
    return pl.pallas_call(
        _tc_pre_kernel,
        grid=(grid,),
        in_specs=[
            pl.BlockSpec((blk, IN_C), lambda i: (i, 0)),
            pl.BlockSpec((IN_C, NA * OUT_C), lambda i: (0, 0)),
            pl.BlockSpec((IN_C, OUT_C), lambda i: (0, 0)),
        ],
        out_specs=[
            pl.BlockSpec((blk, NA * OUT_C), lambda i: (i, 0)),
            pl.BlockSpec((blk, OUT_C), lambda i: (i, 0)),
        ],
        out_shape=[
            jax.ShapeDtypeStruct((N, NA * OUT_C), jnp.float32),
            jax.ShapeDtypeStruct((N, OUT_C), jnp.float32),
        ],
    )(x, zw, uw)


def _tc_combine_kernel(x_ref, g2_ref, sxu_ref, cnt_ref, sab_ref, deg_ref,
                       sea_ref, w1_ref, ws2_ref, w3_ref, ws3_ref, b_ref,
                       out_ref):
    xb = x_ref[...]
    cnt = cnt_ref[...]
    sab = sab_ref[...]
    deg = deg_ref[...]
    sea = sea_ref[...]
    acc = jnp.dot(xb, ws2_ref[...], preferred_element_type=jnp.float32)
    acc = acc + g2_ref[...] + b_ref[...]
    for b in range(NA):
        yb = jnp.dot(xb, w1_ref[..., b * OUT_C:(b + 1) * OUT_C],
                     preferred_element_type=jnp.float32)
        acc = acc + cnt[:, b:b + 1] * yb
        acc = acc + sab[:, b:b + 1] * w3_ref[b:b + 1, :]
    out_ref[...] = sxu_ref[...] + sea * ws3_ref[...] + deg * acc


def _tc_combine(x, g2, sxu, cnt, sab, deg, sea, w1, ws2t, w3t, ws3, bvec):
    blk = 1000
    grid = N // blk
    return pl.pallas_call(
        _tc_combine_kernel,
        grid=(grid,),
        in_specs=[
            pl.BlockSpec((blk, IN_C), lambda i: (i, 0)),
            pl.BlockSpec((blk, OUT_C), lambda i: (i, 0)),
            pl.BlockSpec((blk, OUT_C), lambda i: (i, 0)),
            pl.BlockSpec((blk, NA), lambda i: (i, 0)),
            pl.BlockSpec((blk, NA), lambda i: (i, 0)),
            pl.BlockSpec((blk, 1), lambda i: (i, 0)),
            pl.BlockSpec((blk, 1), lambda i: (i, 0)),
            pl.BlockSpec((IN_C, NA * OUT_C), lambda i: (0, 0)),
            pl.BlockSpec((IN_C, OUT_C), lambda i: (0, 0)),
            pl.BlockSpec((NA, OUT_C), lambda i: (0, 0)),
            pl.BlockSpec((1, OUT_C), lambda i: (0, 0)),
            pl.BlockSpec((1, OUT_C), lambda i: (0, 0)),
        ],
        out_specs=pl.BlockSpec((blk, OUT_C), lambda i: (i, 0)),
        out_shape=jax.ShapeDtypeStruct((N, OUT_C), jnp.float32),
    )(x, g2, sxu, cnt, sab, deg, sea, w1, ws2t, w3t, ws3, bvec)


def kernel(x, edge_index, edge_attr, pos, W_message, b_message):
    Wr = W_message.reshape(OUT_C, NA, 2, FD)
    Ws = jnp.sum(Wr[:, :, 0, :], axis=1)
    ws1t = Ws[:, :IN_C].T
    ws2t = Ws[:, IN_C:2 * IN_C].T
    ws3 = Ws[:, 2 * IN_C].reshape(1, OUT_C)
    W2 = Wr[:, :, 1, IN_C:2 * IN_C]
    zw = jnp.transpose(W2, (2, 1, 0)).reshape(IN_C, NA * OUT_C)
    W1 = Wr[:, :, 1, :IN_C]
    w1 = jnp.transpose(W1, (2, 1, 0)).reshape(IN_C, NA * OUT_C)
    w3t = Wr[:, :, 1, 2 * IN_C].T
    bvec = b_message.reshape(1, OUT_C)

    ea_flat = edge_attr.reshape(E)
    px = pos[:, 0]
    py = pos[:, 1]
    pz = pos[:, 2]
    ea_bits = lax.bitcast_convert_type(ea_flat, jnp.int32)
    pk = jnp.stack(
        [edge_index[0].reshape(E // CHUNK, CHUNK),
         edge_index[1].reshape(E // CHUNK, CHUNK),
         ea_bits.reshape(E // CHUNK, CHUNK)], axis=1)

    z_tab, u_tab = _tc_pre(x, zw, ws1t)
    z_tab = z_tab.reshape(N * NA, OUT_C)

    z2d = jnp.zeros((ROWS_N, OUT_C), jnp.float32)
    z1d = jnp.zeros((CNT_PAD // NS,), jnp.float32)
    g2, sxu, cnt, sab, deg, sea = _sc_edge_built()(
        pk, px, py, pz, z_tab, u_tab, z2d, z1d)

    g2 = g2[:N]
    sxu = sxu[:N]
    cnt = cnt[:N * NA].reshape(N, NA)
    sab = sab[:N * NA].reshape(N, NA)
    deg = deg[:N].reshape(N, 1)
    sea = sea[:N].reshape(N, 1)

    return _tc_combine(x, g2, sxu, cnt, sab, deg, sea, w1, ws2t, w3t, ws3, bvec)


# speedup vs baseline: 17.7920x; 1.0679x over previous
import functools
import math

import jax
import jax.numpy as jnp
from jax import lax
from jax.experimental import pallas as pl
from jax.experimental.pallas import tpu as pltpu
from jax.experimental.pallas import tpu_sc as plsc

N = 10000
E = 160000
IN_C = 128
OUT_C = 128
NA = 7
FD = 2 * IN_C + 1

NS = 16
CHUNK = 128
E_PAD = 161792
EPT = E_PAD // NS
NCHUNK = EPT // CHUNK
CNT_PAD = 71680
DEG_PAD = 10240
ROWS_N = DEG_PAD // NS

_T2 = [float(math.cos(k * math.pi / NA) ** 2) for k in range(1, NA)]


def _sc_edge_kernel(pk_ref, px_ref, py_ref, pz_ref, z_ref, u_ref,
                    z2d_ref, z1d_ref,
                    g2_out, sxu_out, cnt_out, sab_out, deg_out, sea_out,
                    acc, s1, s2,
                    pk0, pk1, ea0, ea1, zidx0, zidx1, ridx0, ridx1,
                    pxr0, pyr0, pzr0, pxc0, pyc0, pzc0,
                    pxr1, pyr1, pzr1, pxc1, pyc1, pzc1,
                    rows0, rows1, ones_v,
                    sem_i0, sem_i1, sem_p0, sem_p1, sem_z0, sem_z1, sem_s):
    cid = lax.axis_index("c")
    sid = lax.axis_index("s")

    pltpu.sync_copy(z2d_ref, acc.at[pl.ds(sid * ROWS_N, ROWS_N), :])
    pltpu.sync_copy(z1d_ref.at[pl.ds(0, CNT_PAD // NS)],
                    s1.at[pl.ds(sid * (CNT_PAD // NS), CNT_PAD // NS)])
    pltpu.sync_copy(z1d_ref.at[pl.ds(0, CNT_PAD // NS)],
                    s2.at[pl.ds(sid * (CNT_PAD // NS), CNT_PAD // NS)])

    for g in range(CHUNK // 16):
        ones_v[pl.ds(g * 16, 16)] = jnp.ones((16,), jnp.float32)

    plsc.subcore_barrier()

    cbase = sid * NCHUNK
    B = ((pk0, ea0, zidx0, ridx0, (pxr0, pyr0, pzr0, pxc0, pyc0, pzc0),
          rows0, sem_p0, sem_z0),
         (pk1, ea1, zidx1, ridx1, (pxr1, pyr1, pzr1, pxc1, pyc1, pzc1),
          rows1, sem_p1, sem_z1))

    def issue_pos(b):
        pk, _, _, _, pos, _, sem_p, _ = B[b]
        pxr, pyr, pzr, pxc, pyc, pzc = pos
        return (pltpu.async_copy(px_ref.at[pk.at[0]], pxr, sem_p),
                pltpu.async_copy(py_ref.at[pk.at[0]], pyr, sem_p),
                pltpu.async_copy(pz_ref.at[pk.at[0]], pzr, sem_p),
                pltpu.async_copy(px_ref.at[pk.at[1]], pxc, sem_p),
                pltpu.async_copy(py_ref.at[pk.at[1]], pyc, sem_p),
                pltpu.async_copy(pz_ref.at[pk.at[1]], pzc, sem_p))

    def unpack_ea(b):
        pk, ea_v, _, _, _, _, _, _ = B[b]
        for g in range(CHUNK // 16):
            bits = pk[2, pl.ds(g * 16, 16)]
            ea_v[pl.ds(g * 16, 16)] = plsc.bitcast(bits, jnp.float32)

    def compute_bins(b):
        pk, _, zidx_v, ridx_v, pos, _, _, _ = B[b]
        pxr, pyr, pzr, pxc, pyc, pzc = pos
        for g in range(CHUNK // 16):
            r16 = pk[0, pl.ds(g * 16, 16)]
            c16 = pk[1, pl.ds(g * 16, 16)]
            dx = pxc[pl.ds(g * 16, 16)] - pxr[pl.ds(g * 16, 16)]
            dy = pyc[pl.ds(g * 16, 16)] - pyr[pl.ds(g * 16, 16)]
            dz = pzc[pl.ds(g * 16, 16)] - pzr[pl.ds(g * 16, 16)]
            s = dx * dx + dy * dy + dz * dz
            vx2 = dx * dx
            neg = dx < 0.0
            bins = jnp.zeros((16,), jnp.int32)
            for k in range(NA - 1):
                if k < 3:
                    hit = neg | (vx2 < _T2[k] * s)
                else:
                    hit = neg & (vx2 > _T2[k] * s)
                bins = bins + hit.astype(jnp.int32)
            bins = jnp.where(s == 0.0, 3, bins)
            zidx_v[pl.ds(g * 16, 16)] = c16 * NA + bins
            ridx_v[pl.ds(g * 16, 16)] = r16 * NA + bins

    def scatter_c0(b):
        pk, ea_v, _, ridx_v, _, rows_v, _, _ = B[b]
        return (pltpu.async_copy(rows_v, acc.at[pk.at[0]], sem_s, add=True),
                pltpu.async_copy(ones_v, s1.at[ridx_v], sem_s, add=True),
                pltpu.async_copy(ea_v, s2.at[ridx_v], sem_s, add=True))

    def scatter_c1(b):
        pk, ea_v, _, _, _, rows_v, _, _ = B[b]
        return (pltpu.async_copy(rows_v, acc.at[pk.at[1]], sem_s, add=True),
                pltpu.async_copy(ones_v, s1.at[pk.at[1]], sem_s, add=True),
                pltpu.async_copy(ea_v, s2.at[pk.at[1]], sem_s, add=True))

    def do_c0(la, lb, tail):
        la.wait()
        pa = issue_pos(0)
        if not tail:
            lb.wait()
            pb = issue_pos(1)
        for c in pa:
            c.wait()
        compute_bins(0)
        za = pltpu.async_copy(z_ref.at[zidx0], rows0, sem_z0)
        unpack_ea(0)
        if not tail:
            for c in pb:
                c.wait()
            compute_bins(1)
            zb = pltpu.async_copy(z_ref.at[zidx1], rows1, sem_z1)
            unpack_ea(1)
        za.wait()
        sa = scatter_c0(0)
        if not tail:
            zb.wait()
            sb = scatter_c0(1)
        for c in sa:
            c.wait()
        if not tail:
            for c in sb:
                c.wait()

    def do_c1(la, lb, tail):
        la.wait()
        ua = pltpu.async_copy(u_ref.at[pk0.at[0]], rows0, sem_z0)
        if not tail:
            lb.wait()
            ub = pltpu.async_copy(u_ref.at[pk1.at[0]], rows1, sem_z1)
        ua.wait()
        unpack_ea(0)
        sa = scatter_c1(0)
        if not tail:
            ub.wait()
            unpack_ea(1)
            sb = scatter_c1(1)
        for c in sa:
            c.wait()
        if not tail:
            for c in sb:
                c.wait()

    def pair_body(i, carry):
        la = pltpu.async_copy(pk_ref.at[cbase + 2 * i], pk0, sem_i0)
        lb = pltpu.async_copy(pk_ref.at[cbase + 2 * i + 1], pk1, sem_i1)

        @pl.when(cid == 0)
        def _c0():
            do_c0(la, lb, False)

        @pl.when(cid == 1)
        def _c1():
            do_c1(la, lb, False)

        return carry

    lax.fori_loop(0, NCHUNK // 2, pair_body, 0)

    if NCHUNK % 2:
        lt = pltpu.async_copy(pk_ref.at[cbase + NCHUNK - 1], pk0, sem_i0)

        @pl.when(cid == 0)
        def _t0():
            do_c0(lt, None, True)

        @pl.when(cid == 1)
        def _t1():
            do_c1(lt, None, True)

    plsc.subcore_barrier()

    @pl.when(cid == 0)
    def _out0():
        pltpu.sync_copy(acc.at[pl.ds(sid * ROWS_N, ROWS_N), :],
                        g2_out.at[pl.ds(sid * ROWS_N, ROWS_N), :])
        pltpu.sync_copy(s1.at[pl.ds(sid * (CNT_PAD // NS), CNT_PAD // NS)],
                        cnt_out.at[pl.ds(sid * (CNT_PAD // NS), CNT_PAD // NS)])
        pltpu.sync_copy(s2.at[pl.ds(sid * (CNT_PAD // NS), CNT_PAD // NS)],
                        sab_out.at[pl.ds(sid * (CNT_PAD // NS), CNT_PAD // NS)])

    @pl.when(cid == 1)
    def _out1():
        pltpu.sync_copy(acc.at[pl.ds(sid * ROWS_N, ROWS_N), :],
                        sxu_out.at[pl.ds(sid * ROWS_N, ROWS_N), :])
        pltpu.sync_copy(s1.at[pl.ds(sid * (DEG_PAD // NS), DEG_PAD // NS)],
                        deg_out.at[pl.ds(sid * (DEG_PAD // NS), DEG_PAD // NS)])
        pltpu.sync_copy(s2.at[pl.ds(sid * (DEG_PAD // NS), DEG_PAD // NS)],
                        sea_out.at[pl.ds(sid * (DEG_PAD // NS), DEG_PAD // NS)])


@functools.lru_cache(maxsize=1)
def _sc_edge_built():
    return functools.partial(
        pl.kernel,
        out_type=[
            jax.ShapeDtypeStruct((DEG_PAD, OUT_C), jnp.float32),
            jax.ShapeDtypeStruct((DEG_PAD, OUT_C), jnp.float32),
            jax.ShapeDtypeStruct((CNT_PAD,), jnp.float32),
            jax.ShapeDtypeStruct((CNT_PAD,), jnp.float32),
            jax.ShapeDtypeStruct((DEG_PAD,), jnp.float32),
            jax.ShapeDtypeStruct((DEG_PAD,), jnp.float32),
        ],
        mesh=plsc.VectorSubcoreMesh(core_axis_name="c", subcore_axis_name="s",
                                    num_cores=2, num_subcores=NS),
        compiler_params=pltpu.CompilerParams(needs_layout_passes=False),
        scratch_types=[
            pltpu.VMEM_SHARED((DEG_PAD, OUT_C), jnp.float32),
            pltpu.VMEM_SHARED((CNT_PAD,), jnp.float32),
            pltpu.VMEM_SHARED((CNT_PAD,), jnp.float32),
            pltpu.VMEM((3, CHUNK), jnp.int32),
            pltpu.VMEM((3, CHUNK), jnp.int32),
            pltpu.VMEM((CHUNK,), jnp.float32),
            pltpu.VMEM((CHUNK,), jnp.float32),
            pltpu.VMEM((CHUNK,), jnp.int32),
            pltpu.VMEM((CHUNK,), jnp.int32),
            pltpu.VMEM((CHUNK,), jnp.int32),
            pltpu.VMEM((CHUNK,), jnp.int32),
            pltpu.VMEM((CHUNK,), jnp.float32),
            pltpu.VMEM((CHUNK,), jnp.float32),
            pltpu.VMEM((CHUNK,), jnp.float32),
            pltpu.VMEM((CHUNK,), jnp.float32),
            pltpu.VMEM((CHUNK,), jnp.float32),
            pltpu.VMEM((CHUNK,), jnp.float32),
            pltpu.VMEM((CHUNK,), jnp.float32),
            pltpu.VMEM((CHUNK,), jnp.float32),
            pltpu.VMEM((CHUNK,), jnp.float32),
            pltpu.VMEM((CHUNK,), jnp.float32),
            pltpu.VMEM((CHUNK,), jnp.float32),
            pltpu.VMEM((CHUNK,), jnp.float32),
            pltpu.VMEM((CHUNK, OUT_C), jnp.float32),
            pltpu.VMEM((CHUNK, OUT_C), jnp.float32),
            pltpu.VMEM((CHUNK,), jnp.float32),
            pltpu.SemaphoreType.DMA,
            pltpu.SemaphoreType.DMA,
            pltpu.SemaphoreType.DMA,
            pltpu.SemaphoreType.DMA,
            pltpu.SemaphoreType.DMA,
            pltpu.SemaphoreType.DMA,
            pltpu.SemaphoreType.DMA,
        ],
    )(_sc_edge_kernel)


def _tc_pre_kernel(x_ref, zw_ref, uw_ref, z_out, u_out):
    xb = x_ref[...]
    z_out[...] = jnp.dot(xb, zw_ref[...], preferred_element_type=jnp.float32)
    u_out[...] = jnp.dot(xb, uw_ref[...], preferred_element_type=jnp.float32)


def _tc_pre(x, zw, uw):
    blk = 1024
    grid = DEG_PAD // blk
    return pl.pallas_call(
        _tc_pre_kernel,
        grid=(grid,),
        in_specs=[
            pl.BlockSpec((blk, IN_C), lambda i: (i, 0)),
            pl.BlockSpec((IN_C, NA * OUT_C), lambda i: (0, 0)),
            pl.BlockSpec((IN_C, OUT_C), lambda i: (0, 0)),
        ],
        out_specs=[
            pl.BlockSpec((blk, NA * OUT_C), lambda i: (i, 0)),
            pl.BlockSpec((blk, OUT_C), lambda i: (i, 0)),
        ],
        out_shape=[
            jax.ShapeDtypeStruct((DEG_PAD, NA * OUT_C), jnp.float32),
            jax.ShapeDtypeStruct((DEG_PAD, OUT_C), jnp.float32),
        ],
    )(x, zw, uw)


def _tc_combine_kernel(x_ref, g2_ref, sxu_ref, cnt_ref, sab_ref, deg_ref,
                       sea_ref, w1_ref, ws2_ref, w3_ref, ws3_ref, b_ref,
                       out_ref):
    xb = x_ref[...]
    cnt = cnt_ref[...]
    sab = sab_ref[...]
    deg = deg_ref[...]
    sea = sea_ref[...]
    acc = jnp.dot(xb, ws2_ref[...], preferred_element_type=jnp.float32)
    acc = acc + g2_ref[...] + b_ref[...]
    for b in range(NA):
        yb = jnp.dot(xb, w1_ref[..., b * OUT_C:(b + 1) * OUT_C],
                     preferred_element_type=jnp.float32)
        acc = acc + cnt[:, b:b + 1] * yb
        acc = acc + sab[:, b:b + 1] * w3_ref[b:b + 1, :]
    out_ref[...] = sxu_ref[...] + sea * ws3_ref[...] + deg * acc


def _tc_combine(x, g2, sxu, cnt, sab, deg, sea, w1, ws2t, w3t, ws3, bvec):
    blk = 1000
    grid = N // blk
    return pl.pallas_call(
        _tc_combine_kernel,
        grid=(grid,),
        in_specs=[
            pl.BlockSpec((blk, IN_C), lambda i: (i, 0)),
            pl.BlockSpec((blk, OUT_C), lambda i: (i, 0)),
            pl.BlockSpec((blk, OUT_C), lambda i: (i, 0)),
            pl.BlockSpec((blk, NA), lambda i: (i, 0)),
            pl.BlockSpec((blk, NA), lambda i: (i, 0)),
            pl.BlockSpec((blk, 1), lambda i: (i, 0)),
            pl.BlockSpec((blk, 1), lambda i: (i, 0)),
            pl.BlockSpec((IN_C, NA * OUT_C), lambda i: (0, 0)),
            pl.BlockSpec((IN_C, OUT_C), lambda i: (0, 0)),
            pl.BlockSpec((NA, OUT_C), lambda i: (0, 0)),
            pl.BlockSpec((1, OUT_C), lambda i: (0, 0)),
            pl.BlockSpec((1, OUT_C), lambda i: (0, 0)),
        ],
        out_specs=pl.BlockSpec((blk, OUT_C), lambda i: (i, 0)),
        out_shape=jax.ShapeDtypeStruct((N, OUT_C), jnp.float32),
    )(x, g2, sxu, cnt, sab, deg, sea, w1, ws2t, w3t, ws3, bvec)


def kernel(x, edge_index, edge_attr, pos, W_message, b_message):
    Wr = W_message.reshape(OUT_C, NA, 2, FD)
    Ws = jnp.sum(Wr[:, :, 0, :], axis=1)
    ws1t = Ws[:, :IN_C].T
    ws2t = Ws[:, IN_C:2 * IN_C].T
    ws3 = Ws[:, 2 * IN_C].reshape(1, OUT_C)
    W2 = Wr[:, :, 1, IN_C:2 * IN_C]
    zw = jnp.transpose(W2, (2, 1, 0)).reshape(IN_C, NA * OUT_C)
    W1 = Wr[:, :, 1, :IN_C]
    w1 = jnp.transpose(W1, (2, 1, 0)).reshape(IN_C, NA * OUT_C)
    w3t = Wr[:, :, 1, 2 * IN_C].T
    bvec = b_message.reshape(1, OUT_C)

    ea_flat = edge_attr.reshape(E)
    posp = jnp.pad(pos, ((0, DEG_PAD - N), (0, 0)))
    px = posp[:, 0]
    py = posp[:, 1]
    pz = posp[:, 2]
    ea_bits = lax.bitcast_convert_type(ea_flat, jnp.int32)
    npad = E_PAD - E
    pad_idx = N + (jnp.arange(npad, dtype=jnp.int32) % (DEG_PAD - N))
    rowp = jnp.concatenate([edge_index[0], pad_idx])
    colp = jnp.concatenate([edge_index[1], pad_idx])
    eap = jnp.concatenate([ea_bits, jnp.zeros((npad,), jnp.int32)])
    pk = jnp.stack(
        [rowp.reshape(E_PAD // CHUNK, CHUNK),
         colp.reshape(E_PAD // CHUNK, CHUNK),
         eap.reshape(E_PAD // CHUNK, CHUNK)], axis=1)

    xp = jnp.pad(x, ((0, DEG_PAD - N), (0, 0)))
    z_tab, u_tab = _tc_pre(xp, zw, ws1t)
    z_tab = z_tab.reshape(DEG_PAD * NA, OUT_C)

    z2d = jnp.zeros((ROWS_N, OUT_C), jnp.float32)
    z1d = jnp.zeros((CNT_PAD // NS,), jnp.float32)
    g2, sxu, cnt, sab, deg, sea = _sc_edge_built()(
        pk, px, py, pz, z_tab, u_tab, z2d, z1d)

    g2 = g2[:N]
    sxu = sxu[:N]
    cnt = cnt[:N * NA].reshape(N, NA)
    sab = sab[:N * NA].reshape(N, NA)
    deg = deg[:N].reshape(N, 1)
    sea = sea[:N].reshape(N, 1)

    return _tc_combine(x, g2, sxu, cnt, sab, deg, sea, w1, ws2t, w3t, ws3, bvec)


# speedup vs baseline: 17.9929x; 1.0113x over previous
import functools
import math

import jax
import jax.numpy as jnp
from jax import lax
from jax.experimental import pallas as pl
from jax.experimental.pallas import tpu as pltpu
from jax.experimental.pallas import tpu_sc as plsc

N = 10000
E = 160000
IN_C = 128
OUT_C = 128
NA = 7
FD = 2 * IN_C + 1

NS = 16
CHUNK = 80
NBUF = 3
E_PAD = 160000
EPT = E_PAD // NS
NCHUNK = EPT // CHUNK
CNT_PAD = 71680
DEG_PAD = 10240
ROWS_N = DEG_PAD // NS

_T2 = [float(math.cos(k * math.pi / NA) ** 2) for k in range(1, NA)]


def _sc_edge_kernel(pk_ref, px_ref, py_ref, pz_ref, z_ref, u_ref,
                    z2d_ref, z1d_ref,
                    g2_out, sxu_out, cnt_out, sab_out, deg_out, sea_out,
                    acc, s1, s2, *scr):
    pk = scr[0:NBUF]
    ea = scr[NBUF:2 * NBUF]
    zidx = scr[2 * NBUF:3 * NBUF]
    ridx = scr[3 * NBUF:4 * NBUF]
    pos = [scr[4 * NBUF + 6 * b:4 * NBUF + 6 * b + 6] for b in range(NBUF)]
    rows = scr[10 * NBUF:11 * NBUF]
    ones_v = scr[11 * NBUF]
    sem_i = scr[11 * NBUF + 1:12 * NBUF + 1]
    sem_p = scr[12 * NBUF + 1:13 * NBUF + 1]
    sem_z = scr[13 * NBUF + 1:14 * NBUF + 1]
    sem_s = scr[14 * NBUF + 1]
    cid = lax.axis_index("c")
    sid = lax.axis_index("s")

    pltpu.sync_copy(z2d_ref, acc.at[pl.ds(sid * ROWS_N, ROWS_N), :])
    pltpu.sync_copy(z1d_ref.at[pl.ds(0, CNT_PAD // NS)],
                    s1.at[pl.ds(sid * (CNT_PAD // NS), CNT_PAD // NS)])
    pltpu.sync_copy(z1d_ref.at[pl.ds(0, CNT_PAD // NS)],
                    s2.at[pl.ds(sid * (CNT_PAD // NS), CNT_PAD // NS)])

    for g in range(CHUNK // 16):
        ones_v[pl.ds(g * 16, 16)] = jnp.ones((16,), jnp.float32)

    plsc.subcore_barrier()

    cbase = sid * NCHUNK

    def issue_pos(b):
        pkb = pk[b]
        pxr, pyr, pzr, pxc, pyc, pzc = pos[b]
        return (pltpu.async_copy(px_ref.at[pkb.at[0]], pxr, sem_p[b]),
                pltpu.async_copy(py_ref.at[pkb.at[0]], pyr, sem_p[b]),
                pltpu.async_copy(pz_ref.at[pkb.at[0]], pzr, sem_p[b]),
                pltpu.async_copy(px_ref.at[pkb.at[1]], pxc, sem_p[b]),
                pltpu.async_copy(py_ref.at[pkb.at[1]], pyc, sem_p[b]),
                pltpu.async_copy(pz_ref.at[pkb.at[1]], pzc, sem_p[b]))

    def unpack_ea(b):
        pkb, ea_v = pk[b], ea[b]
        for g in range(CHUNK // 16):
            bits = pkb[2, pl.ds(g * 16, 16)]
            ea_v[pl.ds(g * 16, 16)] = plsc.bitcast(bits, jnp.float32)

    def compute_bins(b):
        pkb, zidx_v, ridx_v = pk[b], zidx[b], ridx[b]
        pxr, pyr, pzr, pxc, pyc, pzc = pos[b]
        for g in range(CHUNK // 16):
            r16 = pkb[0, pl.ds(g * 16, 16)]
            c16 = pkb[1, pl.ds(g * 16, 16)]
            dx = pxc[pl.ds(g * 16, 16)] - pxr[pl.ds(g * 16, 16)]
            dy = pyc[pl.ds(g * 16, 16)] - pyr[pl.ds(g * 16, 16)]
            dz = pzc[pl.ds(g * 16, 16)] - pzr[pl.ds(g * 16, 16)]
            s = dx * dx + dy * dy + dz * dz
            vx2 = dx * dx
            neg = dx < 0.0
            bins = jnp.zeros((16,), jnp.int32)
            for k in range(NA - 1):
                if k < 3:
                    hit = neg | (vx2 < _T2[k] * s)
                else:
                    hit = neg & (vx2 > _T2[k] * s)
                bins = bins + hit.astype(jnp.int32)
            bins = jnp.where(s == 0.0, 3, bins)
            zidx_v[pl.ds(g * 16, 16)] = c16 * NA + bins
            ridx_v[pl.ds(g * 16, 16)] = r16 * NA + bins

    def scatter_c0(b):
        return (pltpu.async_copy(rows[b], acc.at[pk[b].at[0]], sem_s, add=True),
                pltpu.async_copy(ones_v, s1.at[ridx[b]], sem_s, add=True),
                pltpu.async_copy(ea[b], s2.at[ridx[b]], sem_s, add=True))

    def scatter_c1(b):
        return (pltpu.async_copy(rows[b], acc.at[pk[b].at[1]], sem_s, add=True),
                pltpu.async_copy(ones_v, s1.at[pk[b].at[1]], sem_s, add=True),
                pltpu.async_copy(ea[b], s2.at[pk[b].at[1]], sem_s, add=True))

    def do_c0(ls):
        nact = len(ls)
        ps = []
        for b in range(nact):
            ls[b].wait()
            ps.append(issue_pos(b))
        zs = []
        for b in range(nact):
            for c in ps[b]:
                c.wait()
            compute_bins(b)
            zs.append(pltpu.async_copy(z_ref.at[zidx[b]], rows[b], sem_z[b]))
            unpack_ea(b)
        ss = []
        for b in range(nact):
            zs[b].wait()
            ss.extend(scatter_c0(b))
        for c in ss:
            c.wait()

    def do_c1(ls):
        nact = len(ls)
        us = []
        for b in range(nact):
            ls[b].wait()
            us.append(pltpu.async_copy(u_ref.at[pk[b].at[0]], rows[b], sem_z[b]))
        ss = []
        for b in range(nact):
            us[b].wait()
            unpack_ea(b)
            ss.extend(scatter_c1(b))
        for c in ss:
            c.wait()

    def grp_body(i, carry):
        ls = [pltpu.async_copy(pk_ref.at[cbase + NBUF * i + b], pk[b], sem_i[b])
              for b in range(NBUF)]

        @pl.when(cid == 0)
        def _c0():
            do_c0(ls)

        @pl.when(cid == 1)
        def _c1():
            do_c1(ls)

        return carry

    lax.fori_loop(0, NCHUNK // NBUF, grp_body, 0)

    _TAIL = NCHUNK % NBUF
    if _TAIL:
        lt = [pltpu.async_copy(pk_ref.at[cbase + NCHUNK - _TAIL + b], pk[b],
                               sem_i[b]) for b in range(_TAIL)]

        @pl.when(cid == 0)
        def _t0():
            do_c0(lt)

        @pl.when(cid == 1)
        def _t1():
            do_c1(lt)

    plsc.subcore_barrier()

    @pl.when(cid == 0)
    def _out0():
        pltpu.sync_copy(acc.at[pl.ds(sid * ROWS_N, ROWS_N), :],
                        g2_out.at[pl.ds(sid * ROWS_N, ROWS_N), :])
        pltpu.sync_copy(s1.at[pl.ds(sid * (CNT_PAD // NS), CNT_PAD // NS)],
                        cnt_out.at[pl.ds(sid * (CNT_PAD // NS), CNT_PAD // NS)])
        pltpu.sync_copy(s2.at[pl.ds(sid * (CNT_PAD // NS), CNT_PAD // NS)],
                        sab_out.at[pl.ds(sid * (CNT_PAD // NS), CNT_PAD // NS)])

    @pl.when(cid == 1)
    def _out1():
        pltpu.sync_copy(acc.at[pl.ds(sid * ROWS_N, ROWS_N), :],
                        sxu_out.at[pl.ds(sid * ROWS_N, ROWS_N), :])
        pltpu.sync_copy(s1.at[pl.ds(sid * (DEG_PAD // NS), DEG_PAD // NS)],
                        deg_out.at[pl.ds(sid * (DEG_PAD // NS), DEG_PAD // NS)])
        pltpu.sync_copy(s2.at[pl.ds(sid * (DEG_PAD // NS), DEG_PAD // NS)],
                        sea_out.at[pl.ds(sid * (DEG_PAD // NS), DEG_PAD // NS)])


@functools.lru_cache(maxsize=1)
def _sc_edge_built():
    return functools.partial(
        pl.kernel,
        out_type=[
            jax.ShapeDtypeStruct((DEG_PAD, OUT_C), jnp.float32),
            jax.ShapeDtypeStruct((DEG_PAD, OUT_C), jnp.float32),
            jax.ShapeDtypeStruct((CNT_PAD,), jnp.float32),
            jax.ShapeDtypeStruct((CNT_PAD,), jnp.float32),
            jax.ShapeDtypeStruct((DEG_PAD,), jnp.float32),
            jax.ShapeDtypeStruct((DEG_PAD,), jnp.float32),
        ],
        mesh=plsc.VectorSubcoreMesh(core_axis_name="c", subcore_axis_name="s",
                                    num_cores=2, num_subcores=NS),
        compiler_params=pltpu.CompilerParams(needs_layout_passes=False),
        scratch_types=[
            pltpu.VMEM_SHARED((DEG_PAD, OUT_C), jnp.float32),
            pltpu.VMEM_SHARED((CNT_PAD,), jnp.float32),
            pltpu.VMEM_SHARED((CNT_PAD,), jnp.float32),
        ] + [pltpu.VMEM((3, CHUNK), jnp.int32)] * NBUF
          + [pltpu.VMEM((CHUNK,), jnp.float32)] * NBUF
          + [pltpu.VMEM((CHUNK,), jnp.int32)] * NBUF
          + [pltpu.VMEM((CHUNK,), jnp.int32)] * NBUF
          + [pltpu.VMEM((CHUNK,), jnp.float32)] * (6 * NBUF)
          + [pltpu.VMEM((CHUNK, OUT_C), jnp.float32)] * NBUF
          + [pltpu.VMEM((CHUNK,), jnp.float32)]
          + [pltpu.SemaphoreType.DMA] * (3 * NBUF + 1),
    )(_sc_edge_kernel)


def _tc_pre_kernel(x_ref, zw_ref, uw_ref, z_out, u_out):
    xb = x_ref[...]
    z_out[...] = jnp.dot(xb, zw_ref[...], preferred_element_type=jnp.float32)
    u_out[...] = jnp.dot(xb, uw_ref[...], preferred_element_type=jnp.float32)


def _tc_pre(x, zw, uw):
    blk = 1024
    grid = DEG_PAD // blk
    return pl.pallas_call(
        _tc_pre_kernel,
        grid=(grid,),
        in_specs=[
            pl.BlockSpec((blk, IN_C), lambda i: (i, 0)),
            pl.BlockSpec((IN_C, NA * OUT_C), lambda i: (0, 0)),
            pl.BlockSpec((IN_C, OUT_C), lambda i: (0, 0)),
        ],
        out_specs=[
            pl.BlockSpec((blk, NA * OUT_C), lambda i: (i, 0)),
            pl.BlockSpec((blk, OUT_C), lambda i: (i, 0)),
        ],
        out_shape=[
            jax.ShapeDtypeStruct((DEG_PAD, NA * OUT_C), jnp.float32),
            jax.ShapeDtypeStruct((DEG_PAD, OUT_C), jnp.float32),
        ],
    )(x, zw, uw)


def _tc_combine_kernel(x_ref, g2_ref, sxu_ref, cnt_ref, sab_ref, deg_ref,
                       sea_ref, w1_ref, ws2_ref, w3_ref, ws3_ref, b_ref,
                       out_ref):
    xb = x_ref[...]
    cnt = cnt_ref[...]
    sab = sab_ref[...]
    deg = deg_ref[...]
    sea = sea_ref[...]
    acc = jnp.dot(xb, ws2_ref[...], preferred_element_type=jnp.float32)
    acc = acc + g2_ref[...] + b_ref[...]
    for b in range(NA):
        yb = jnp.dot(xb, w1_ref[..., b * OUT_C:(b + 1) * OUT_C],
                     preferred_element_type=jnp.float32)
        acc = acc + cnt[:, b:b + 1] * yb
        acc = acc + sab[:, b:b + 1] * w3_ref[b:b + 1, :]
    out_ref[...] = sxu_ref[...] + sea * ws3_ref[...] + deg * acc


def _tc_combine(x, g2, sxu, cnt, sab, deg, sea, w1, ws2t, w3t, ws3, bvec):
    blk = 1000
    grid = N // blk
    return pl.pallas_call(
        _tc_combine_kernel,
        grid=(grid,),
        in_specs=[
            pl.BlockSpec((blk, IN_C), lambda i: (i, 0)),
            pl.BlockSpec((blk, OUT_C), lambda i: (i, 0)),
            pl.BlockSpec((blk, OUT_C), lambda i: (i, 0)),
            pl.BlockSpec((blk, NA), lambda i: (i, 0)),
            pl.BlockSpec((blk, NA), lambda i: (i, 0)),
            pl.BlockSpec((blk, 1), lambda i: (i, 0)),
            pl.BlockSpec((blk, 1), lambda i: (i, 0)),
            pl.BlockSpec((IN_C, NA * OUT_C), lambda i: (0, 0)),
            pl.BlockSpec((IN_C, OUT_C), lambda i: (0, 0)),
            pl.BlockSpec((NA, OUT_C), lambda i: (0, 0)),
            pl.BlockSpec((1, OUT_C), lambda i: (0, 0)),
            pl.BlockSpec((1, OUT_C), lambda i: (0, 0)),
        ],
        out_specs=pl.BlockSpec((blk, OUT_C), lambda i: (i, 0)),
        out_shape=jax.ShapeDtypeStruct((N, OUT_C), jnp.float32),
    )(x, g2, sxu, cnt, sab, deg, sea, w1, ws2t, w3t, ws3, bvec)


def kernel(x, edge_index, edge_attr, pos, W_message, b_message):
    Wr = W_message.reshape(OUT_C, NA, 2, FD)
    Ws = jnp.sum(Wr[:, :, 0, :], axis=1)
    ws1t = Ws[:, :IN_C].T
    ws2t = Ws[:, IN_C:2 * IN_C].T
    ws3 = Ws[:, 2 * IN_C].reshape(1, OUT_C)
    W2 = Wr[:, :, 1, IN_C:2 * IN_C]
    zw = jnp.transpose(W2, (2, 1, 0)).reshape(IN_C, NA * OUT_C)
    W1 = Wr[:, :, 1, :IN_C]
    w1 = jnp.transpose(W1, (2, 1, 0)).reshape(IN_C, NA * OUT_C)
    w3t = Wr[:, :, 1, 2 * IN_C].T
    bvec = b_message.reshape(1, OUT_C)

    ea_flat = edge_attr.reshape(E)
    posp = jnp.pad(pos, ((0, DEG_PAD - N), (0, 0)))
    px = posp[:, 0]
    py = posp[:, 1]
    pz = posp[:, 2]
    ea_bits = lax.bitcast_convert_type(ea_flat, jnp.int32)
    npad = E_PAD - E
    pad_idx = N + (jnp.arange(npad, dtype=jnp.int32) % (DEG_PAD - N))
    rowp = jnp.concatenate([edge_index[0], pad_idx])
    colp = jnp.concatenate([edge_index[1], pad_idx])
    eap = jnp.concatenate([ea_bits, jnp.zeros((npad,), jnp.int32)])
    pk = jnp.stack(
        [rowp.reshape(E_PAD // CHUNK, CHUNK),
         colp.reshape(E_PAD // CHUNK, CHUNK),
         eap.reshape(E_PAD // CHUNK, CHUNK)], axis=1)

    xp = jnp.pad(x, ((0, DEG_PAD - N), (0, 0)))
    z_tab, u_tab = _tc_pre(xp, zw, ws1t)
    z_tab = z_tab.reshape(DEG_PAD * NA, OUT_C)

    z2d = jnp.zeros((ROWS_N, OUT_C), jnp.float32)
    z1d = jnp.zeros((CNT_PAD // NS,), jnp.float32)
    g2, sxu, cnt, sab, deg, sea = _sc_edge_built()(
        pk, px, py, pz, z_tab, u_tab, z2d, z1d)

    g2 = g2[:N]
    sxu = sxu[:N]
    cnt = cnt[:N * NA].reshape(N, NA)
    sab = sab[:N * NA].reshape(N, NA)
    deg = deg[:N].reshape(N, 1)
    sea = sea[:N].reshape(N, 1)

    return _tc_combine(x, g2, sxu, cnt, sab, deg, sea, w1, ws2t, w3t, ws3, bvec)


# speedup vs baseline: 18.2729x; 1.0156x over previous
import functools
import math

import jax
import jax.numpy as jnp
from jax import lax
from jax.experimental import pallas as pl
from jax.experimental.pallas import tpu as pltpu
from jax.experimental.pallas import tpu_sc as plsc

N = 10000
E = 160000
IN_C = 128
OUT_C = 128
NA = 7
FD = 2 * IN_C + 1

NS = 16
CHUNK = 80
NBUF = 3
E_PAD = 160000
EPT = E_PAD // NS
NCHUNK = EPT // CHUNK
CNT_PAD = 71680
DEG_PAD = 10240
ROWS_N = DEG_PAD // NS

_T2 = [float(math.cos(k * math.pi / NA) ** 2) for k in range(1, NA)]


def _sc_edge_kernel(pk_ref, px_ref, py_ref, pz_ref, z_ref, u_ref,
                    z2d_ref, z1d_ref,
                    g2_out, sxu_out, cnt_out, sab_out, deg_out, sea_out,
                    acc, s1, s2, *scr):
    pk = scr[0:NBUF]
    ea = scr[NBUF:2 * NBUF]
    zidx = scr[2 * NBUF:3 * NBUF]
    ridx = scr[3 * NBUF:4 * NBUF]
    pos = [scr[4 * NBUF + 6 * b:4 * NBUF + 6 * b + 6] for b in range(NBUF)]
    rows = scr[10 * NBUF:11 * NBUF]
    ones_v = scr[11 * NBUF]
    sem_i = scr[11 * NBUF + 1:12 * NBUF + 1]
    sem_p = scr[12 * NBUF + 1:13 * NBUF + 1]
    sem_z = scr[13 * NBUF + 1:14 * NBUF + 1]
    sem_s = scr[14 * NBUF + 1]
    cid = lax.axis_index("c")
    sid = lax.axis_index("s")

    pltpu.sync_copy(z2d_ref, acc.at[pl.ds(sid * ROWS_N, ROWS_N), :])
    pltpu.sync_copy(z1d_ref.at[pl.ds(0, CNT_PAD // NS)],
                    s1.at[pl.ds(sid * (CNT_PAD // NS), CNT_PAD // NS)])
    pltpu.sync_copy(z1d_ref.at[pl.ds(0, CNT_PAD // NS)],
                    s2.at[pl.ds(sid * (CNT_PAD // NS), CNT_PAD // NS)])

    for g in range(CHUNK // 16):
        ones_v[pl.ds(g * 16, 16)] = jnp.ones((16,), jnp.float32)

    plsc.subcore_barrier()

    cbase = sid * NCHUNK

    def issue_pos(b):
        pkb = pk[b]
        pxr, pyr, pzr, pxc, pyc, pzc = pos[b]
        return (pltpu.async_copy(px_ref.at[pkb.at[0]], pxr, sem_p[b]),
                pltpu.async_copy(py_ref.at[pkb.at[0]], pyr, sem_p[b]),
                pltpu.async_copy(pz_ref.at[pkb.at[0]], pzr, sem_p[b]),
                pltpu.async_copy(px_ref.at[pkb.at[1]], pxc, sem_p[b]),
                pltpu.async_copy(py_ref.at[pkb.at[1]], pyc, sem_p[b]),
                pltpu.async_copy(pz_ref.at[pkb.at[1]], pzc, sem_p[b]))

    def unpack_ea(b):
        pkb, ea_v = pk[b], ea[b]
        for g in range(CHUNK // 16):
            bits = pkb[2, pl.ds(g * 16, 16)]
            ea_v[pl.ds(g * 16, 16)] = plsc.bitcast(bits, jnp.float32)

    def compute_bins(b):
        pkb, zidx_v, ridx_v = pk[b], zidx[b], ridx[b]
        pxr, pyr, pzr, pxc, pyc, pzc = pos[b]
        for g in range(CHUNK // 16):
            r16 = pkb[0, pl.ds(g * 16, 16)]
            c16 = pkb[1, pl.ds(g * 16, 16)]
            dx = pxc[pl.ds(g * 16, 16)] - pxr[pl.ds(g * 16, 16)]
            dy = pyc[pl.ds(g * 16, 16)] - pyr[pl.ds(g * 16, 16)]
            dz = pzc[pl.ds(g * 16, 16)] - pzr[pl.ds(g * 16, 16)]
            s = dx * dx + dy * dy + dz * dz
            vx2 = dx * dx
            neg = dx < 0.0
            bins = jnp.zeros((16,), jnp.int32)
            for k in range(NA - 1):
                if k < 3:
                    hit = neg | (vx2 < _T2[k] * s)
                else:
                    hit = neg & (vx2 > _T2[k] * s)
                bins = bins + hit.astype(jnp.int32)
            bins = jnp.where(s == 0.0, 3, bins)
            zidx_v[pl.ds(g * 16, 16)] = c16 * NA + bins
            ridx_v[pl.ds(g * 16, 16)] = r16 * NA + bins

    def scatter_c0(b):
        return (pltpu.async_copy(rows[b], acc.at[pk[b].at[0]], sem_s, add=True),
                pltpu.async_copy(ones_v, s1.at[ridx[b]], sem_s, add=True),
                pltpu.async_copy(ea[b], s2.at[ridx[b]], sem_s, add=True))

    def scatter_c1(b):
        return (pltpu.async_copy(rows[b], acc.at[pk[b].at[1]], sem_s, add=True),
                pltpu.async_copy(ones_v, s1.at[pk[b].at[1]], sem_s, add=True),
                pltpu.async_copy(ea[b], s2.at[pk[b].at[1]], sem_s, add=True))

    def do_c0(ls):
        nact = len(ls)
        ps = []
        for b in range(nact):
            ls[b].wait()
            ps.append(issue_pos(b))
        zs = []
        for b in range(nact):
            for c in ps[b]:
                c.wait()
            compute_bins(b)
            zs.append(pltpu.async_copy(z_ref.at[zidx[b]], rows[b], sem_z[b]))
            unpack_ea(b)
        ss = []
        for b in range(nact):
            zs[b].wait()
            ss.extend(scatter_c0(b))
        for c in ss:
            c.wait()

    def do_c1(ls):
        nact = len(ls)
        us = []
        for b in range(nact):
            ls[b].wait()
            us.append(pltpu.async_copy(u_ref.at[pk[b].at[0]], rows[b], sem_z[b]))
        ss = []
        for b in range(nact):
            us[b].wait()
            unpack_ea(b)
            ss.extend(scatter_c1(b))
        for c in ss:
            c.wait()

    def grp_body(i, carry):
        ls = [pltpu.async_copy(pk_ref.at[cbase + NBUF * i + b], pk[b], sem_i[b])
              for b in range(NBUF)]

        @pl.when(cid == 0)
        def _c0():
            do_c0(ls)

        @pl.when(cid == 1)
        def _c1():
            do_c1(ls)

        return carry

    lax.fori_loop(0, NCHUNK // NBUF, grp_body, 0)

    _TAIL = NCHUNK % NBUF
    if _TAIL:
        lt = [pltpu.async_copy(pk_ref.at[cbase + NCHUNK - _TAIL + b], pk[b],
                               sem_i[b]) for b in range(_TAIL)]

        @pl.when(cid == 0)
        def _t0():
            do_c0(lt)

        @pl.when(cid == 1)
        def _t1():
            do_c1(lt)

    plsc.subcore_barrier()

    @pl.when(cid == 0)
    def _out0():
        pltpu.sync_copy(acc.at[pl.ds(sid * ROWS_N, ROWS_N), :],
                        g2_out.at[pl.ds(sid * ROWS_N, ROWS_N), :])
        pltpu.sync_copy(s1.at[pl.ds(sid * (CNT_PAD // NS), CNT_PAD // NS)],
                        cnt_out.at[pl.ds(sid * (CNT_PAD // NS), CNT_PAD // NS)])
        pltpu.sync_copy(s2.at[pl.ds(sid * (CNT_PAD // NS), CNT_PAD // NS)],
                        sab_out.at[pl.ds(sid * (CNT_PAD // NS), CNT_PAD // NS)])

    @pl.when(cid == 1)
    def _out1():
        pltpu.sync_copy(acc.at[pl.ds(sid * ROWS_N, ROWS_N), :],
                        sxu_out.at[pl.ds(sid * ROWS_N, ROWS_N), :])
        pltpu.sync_copy(s1.at[pl.ds(sid * (DEG_PAD // NS), DEG_PAD // NS)],
                        deg_out.at[pl.ds(sid * (DEG_PAD // NS), DEG_PAD // NS)])
        pltpu.sync_copy(s2.at[pl.ds(sid * (DEG_PAD // NS), DEG_PAD // NS)],
                        sea_out.at[pl.ds(sid * (DEG_PAD // NS), DEG_PAD // NS)])


@functools.lru_cache(maxsize=1)
def _sc_edge_built():
    return functools.partial(
        pl.kernel,
        out_type=[
            jax.ShapeDtypeStruct((DEG_PAD, OUT_C), jnp.float32),
            jax.ShapeDtypeStruct((DEG_PAD, OUT_C), jnp.float32),
            jax.ShapeDtypeStruct((CNT_PAD,), jnp.float32),
            jax.ShapeDtypeStruct((CNT_PAD,), jnp.float32),
            jax.ShapeDtypeStruct((DEG_PAD,), jnp.float32),
            jax.ShapeDtypeStruct((DEG_PAD,), jnp.float32),
        ],
        mesh=plsc.VectorSubcoreMesh(core_axis_name="c", subcore_axis_name="s",
                                    num_cores=2, num_subcores=NS),
        compiler_params=pltpu.CompilerParams(needs_layout_passes=False),
        scratch_types=[
            pltpu.VMEM_SHARED((DEG_PAD, OUT_C), jnp.float32),
            pltpu.VMEM_SHARED((CNT_PAD,), jnp.float32),
            pltpu.VMEM_SHARED((CNT_PAD,), jnp.float32),
        ] + [pltpu.VMEM((3, CHUNK), jnp.int32)] * NBUF
          + [pltpu.VMEM((CHUNK,), jnp.float32)] * NBUF
          + [pltpu.VMEM((CHUNK,), jnp.int32)] * NBUF
          + [pltpu.VMEM((CHUNK,), jnp.int32)] * NBUF
          + [pltpu.VMEM((CHUNK,), jnp.float32)] * (6 * NBUF)
          + [pltpu.VMEM((CHUNK, OUT_C), jnp.float32)] * NBUF
          + [pltpu.VMEM((CHUNK,), jnp.float32)]
          + [pltpu.SemaphoreType.DMA] * (3 * NBUF + 1),
    )(_sc_edge_kernel)


def _tc_pre_kernel(x_ref, zuw_ref, z_out, u_out):
    r = jnp.dot(x_ref[...], zuw_ref[...], preferred_element_type=jnp.float32)
    z_out[...] = r[:, :NA * OUT_C]
    u_out[...] = r[:, NA * OUT_C:]


def _tc_pre(x, zuw):
    blk = 1024
    grid = DEG_PAD // blk
    return pl.pallas_call(
        _tc_pre_kernel,
        grid=(grid,),
        in_specs=[
            pl.BlockSpec((blk, IN_C), lambda i: (i, 0)),
            pl.BlockSpec((IN_C, (NA + 1) * OUT_C), lambda i: (0, 0)),
        ],
        out_specs=[
            pl.BlockSpec((blk, NA * OUT_C), lambda i: (i, 0)),
            pl.BlockSpec((blk, OUT_C), lambda i: (i, 0)),
        ],
        out_shape=[
            jax.ShapeDtypeStruct((DEG_PAD, NA * OUT_C), jnp.float32),
            jax.ShapeDtypeStruct((DEG_PAD, OUT_C), jnp.float32),
        ],
    )(x, zuw)


def _tc_combine_kernel(x_ref, g2_ref, sxu_ref, cnt_ref, sab_ref, deg_ref,
                       sea_ref, w12_ref, w3_ref, ws3_ref, b_ref, out_ref):
    cnt = cnt_ref[...]
    deg = deg_ref[...]
    sea = sea_ref[...]
    y = jnp.dot(x_ref[...], w12_ref[...], preferred_element_type=jnp.float32)
    acc = y[:, NA * OUT_C:] + g2_ref[...] + b_ref[...]
    acc = acc + jnp.dot(sab_ref[...], w3_ref[...],
                        preferred_element_type=jnp.float32)
    for b in range(NA):
        acc = acc + cnt[:, b:b + 1] * y[:, b * OUT_C:(b + 1) * OUT_C]
    out_ref[...] = sxu_ref[...] + sea * ws3_ref[...] + deg * acc


def _tc_combine(x, g2, sxu, cnt, sab, deg, sea, w12, w3t, ws3, bvec):
    blk = 1000
    grid = N // blk
    return pl.pallas_call(
        _tc_combine_kernel,
        grid=(grid,),
        in_specs=[
            pl.BlockSpec((blk, IN_C), lambda i: (i, 0)),
            pl.BlockSpec((blk, OUT_C), lambda i: (i, 0)),
            pl.BlockSpec((blk, OUT_C), lambda i: (i, 0)),
            pl.BlockSpec((blk, NA), lambda i: (i, 0)),
            pl.BlockSpec((blk, NA), lambda i: (i, 0)),
            pl.BlockSpec((blk, 1), lambda i: (i, 0)),
            pl.BlockSpec((blk, 1), lambda i: (i, 0)),
            pl.BlockSpec((IN_C, (NA + 1) * OUT_C), lambda i: (0, 0)),
            pl.BlockSpec((NA, OUT_C), lambda i: (0, 0)),
            pl.BlockSpec((1, OUT_C), lambda i: (0, 0)),
            pl.BlockSpec((1, OUT_C), lambda i: (0, 0)),
        ],
        out_specs=pl.BlockSpec((blk, OUT_C), lambda i: (i, 0)),
        out_shape=jax.ShapeDtypeStruct((N, OUT_C), jnp.float32),
    )(x, g2, sxu, cnt, sab, deg, sea, w12, w3t, ws3, bvec)


def kernel(x, edge_index, edge_attr, pos, W_message, b_message):
    Wr = W_message.reshape(OUT_C, NA, 2, FD)
    Ws = jnp.sum(Wr[:, :, 0, :], axis=1)
    ws1t = Ws[:, :IN_C].T
    ws2t = Ws[:, IN_C:2 * IN_C].T
    ws3 = Ws[:, 2 * IN_C].reshape(1, OUT_C)
    W2 = Wr[:, :, 1, IN_C:2 * IN_C]
    zw = jnp.transpose(W2, (2, 1, 0)).reshape(IN_C, NA * OUT_C)
    W1 = Wr[:, :, 1, :IN_C]
    w1 = jnp.transpose(W1, (2, 1, 0)).reshape(IN_C, NA * OUT_C)
    w3t = Wr[:, :, 1, 2 * IN_C].T
    bvec = b_message.reshape(1, OUT_C)

    ea_flat = edge_attr.reshape(E)
    posp = jnp.pad(pos, ((0, DEG_PAD - N), (0, 0)))
    px = posp[:, 0]
    py = posp[:, 1]
    pz = posp[:, 2]
    ea_bits = lax.bitcast_convert_type(ea_flat, jnp.int32)
    npad = E_PAD - E
    pad_idx = N + (jnp.arange(npad, dtype=jnp.int32) % (DEG_PAD - N))
    rowp = jnp.concatenate([edge_index[0], pad_idx])
    colp = jnp.concatenate([edge_index[1], pad_idx])
    eap = jnp.concatenate([ea_bits, jnp.zeros((npad,), jnp.int32)])
    pk = jnp.stack(
        [rowp.reshape(E_PAD // CHUNK, CHUNK),
         colp.reshape(E_PAD // CHUNK, CHUNK),
         eap.reshape(E_PAD // CHUNK, CHUNK)], axis=1)

    xp = jnp.pad(x, ((0, DEG_PAD - N), (0, 0)))
    z_tab, u_tab = _tc_pre(xp, jnp.concatenate([zw, ws1t], axis=1))
    z_tab = z_tab.reshape(DEG_PAD * NA, OUT_C)

    z2d = jnp.zeros((ROWS_N, OUT_C), jnp.float32)
    z1d = jnp.zeros((CNT_PAD // NS,), jnp.float32)
    g2, sxu, cnt, sab, deg, sea = _sc_edge_built()(
        pk, px, py, pz, z_tab, u_tab, z2d, z1d)

    g2 = g2[:N]
    sxu = sxu[:N]
    cnt = cnt[:N * NA].reshape(N, NA)
    sab = sab[:N * NA].reshape(N, NA)
    deg = deg[:N].reshape(N, 1)
    sea = sea[:N].reshape(N, 1)

    w12 = jnp.concatenate([w1, ws2t], axis=1)
    return _tc_combine(x, g2, sxu, cnt, sab, deg, sea, w12, w3t, ws3, bvec)
